# baseline (speedup 1.0000x reference)
import functools

import jax
import jax.numpy as jnp
from jax import lax
from jax.experimental import pallas as pl
from jax.experimental.pallas import tpu as pltpu


def _rup(x, m):
    return (x + m - 1) // m * m


def _matmul_body(x_ref, w_ref, b_ref, o_ref):
    o_ref[...] = (
        jnp.dot(x_ref[...], w_ref[...], preferred_element_type=jnp.float32)
        + b_ref[...]
    )


def _matmul_bias(x, w, b, block_n=1024):
    n, k = x.shape
    m = w.shape[1]
    grid = n // block_n
    return pl.pallas_call(
        _matmul_body,
        grid=(grid,),
        in_specs=[
            pl.BlockSpec((block_n, k), lambda i: (i, 0)),
            pl.BlockSpec((k, m), lambda i: (0, 0)),
            pl.BlockSpec((1, m), lambda i: (0, 0)),
        ],
        out_specs=pl.BlockSpec((block_n, m), lambda i: (i, 0)),
        out_shape=jax.ShapeDtypeStruct((n, m), jnp.float32),
    )(x, w, b)


def _layer_edge_jnp(q, k, v, edge_index, r_ij, We, n, dout):
    src = edge_index[0]
    dst = edge_index[1]
    e = r_ij @ We
    q_i = q[dst]
    k_j = k[src] + e
    v_j = v[src] + e
    alpha = jnp.sum(q_i * k_j, axis=-1) / jnp.sqrt(jnp.float32(dout))
    amax = jax.ops.segment_max(alpha, dst, num_segments=n)
    alpha = jnp.exp(alpha - amax[dst])
    denom = jax.ops.segment_sum(alpha, dst, num_segments=n)
    alpha = alpha / (denom[dst] + 1e-16)
    return jax.ops.segment_sum(alpha[:, None] * v_j, dst, num_segments=n)


def kernel(t, data, edge_index, params):
    n = data.shape[0]
    npad = _rup(n, 1024)
    src = edge_index[0]
    dst = edge_index[1]

    coords = data[:, 0:3]
    d = coords[src] - coords[dst]
    r_ij = jnp.sqrt(jnp.sum(d * d, axis=1) + 1e-12).reshape(-1, 1)

    tt = jnp.ones_like(data[:, :1]) * t
    h = jnp.concatenate([tt, data.astype(jnp.float32)], axis=1)

    n_layers = len(params)
    for l, p in enumerate(params):
        Wq, bq, Wk, bk, Wv, bv, We, Ws, bs = p
        din, dout = Wq.shape
        kp = _rup(din, 8)
        mp = _rup(dout, 128)
        wbig = jnp.zeros((kp, 4 * mp), jnp.float32)
        wbig = wbig.at[:din, 0:dout].set(Wq)
        wbig = wbig.at[:din, mp : mp + dout].set(Wk)
        wbig = wbig.at[:din, 2 * mp : 2 * mp + dout].set(Wv)
        wbig = wbig.at[:din, 3 * mp : 3 * mp + dout].set(Ws)
        bbig = jnp.zeros((1, 4 * mp), jnp.float32)
        bbig = bbig.at[0, 0:dout].set(bq)
        bbig = bbig.at[0, mp : mp + dout].set(bk)
        bbig = bbig.at[0, 2 * mp : 2 * mp + dout].set(bv)
        bbig = bbig.at[0, 3 * mp : 3 * mp + dout].set(bs)
        hp = jnp.zeros((npad, kp), jnp.float32).at[:n, :din].set(h)
        z = _matmul_bias(hp, wbig, bbig)
        q = z[:n, 0:dout]
        kk = z[:n, mp : mp + dout]
        v = z[:n, 2 * mp : 2 * mp + dout]
        s = z[:n, 3 * mp : 3 * mp + dout]
        agg = _layer_edge_jnp(q, kk, v, edge_index, r_ij, We, n, dout)
        h = agg + s
        if l != n_layers - 1:
            h = jax.nn.relu(h)
    return h.astype(jnp.float32)


# speedup vs baseline: 2.4655x; 2.4655x over previous
import functools

import jax
import jax.numpy as jnp
from jax import lax
from jax.experimental import pallas as pl
from jax.experimental.pallas import tpu as pltpu
from jax.experimental.pallas import tpu_sc as plsc

NW = 32
QT = 16
NT = 16


def _rup(x, m):
    return (x + m - 1) // m * m


def _sget(ref, i):
    base = pl.multiple_of((i // 16) * 16, 16)
    v = ref[pl.ds(base, 16)]
    lane = lax.broadcasted_iota(jnp.int32, (16,), 0) == (i - base)
    return jnp.sum(jnp.where(lane, v, jnp.zeros_like(v)), axis=0)


def _proj_body(x_ref, w_ref, b_ref, oq, ok_, ov, os_):
    x = x_ref[...]
    outs = (oq, ok_, ov, os_)
    for i in range(4):
        outs[i][...] = (
            jnp.dot(x, w_ref[i], preferred_element_type=jnp.float32)
            + b_ref[i]
        )


def _proj(x, w4, b4, block_n=1024):
    npad, kp = x.shape
    dp = w4.shape[2]
    grid = npad // block_n
    out_sd = jax.ShapeDtypeStruct((npad, dp), jnp.float32)
    return pl.pallas_call(
        _proj_body,
        grid=(grid,),
        in_specs=[
            pl.BlockSpec((block_n, kp), lambda i: (i, 0)),
            pl.BlockSpec((4, kp, dp), lambda i: (0, 0, 0)),
            pl.BlockSpec((4, 1, dp), lambda i: (0, 0, 0)),
        ],
        out_specs=[pl.BlockSpec((block_n, dp), lambda i: (i, 0))] * 4,
        out_shape=[out_sd] * 4,
    )(x, w4, b4)


def _sc_edge_kernel(n, e_pad, dpu, dpu_mem, wout, ce, inv_sqrt):
    nch = dpu // 16
    mesh = plsc.VectorSubcoreMesh(core_axis_name="c", subcore_axis_name="s")

    def body(q_hbm, k_hbm, v_hbm, src_hbm, dst_hbm, r_hbm, we_hbm,
             nb_hbm, eb_hbm, out_hbm,
             nbv, ebv, wev, idxv, kbuf, vbuf, dbuf, rbuf, qtile, acc,
             otile, sem1, sem2):
        zv = jnp.zeros((16,), jnp.float32)
        wid = lax.axis_index("s") * 2 + lax.axis_index("c")
        pltpu.sync_copy(nb_hbm, nbv)
        pltpu.sync_copy(eb_hbm, ebv)
        pltpu.sync_copy(we_hbm, wev)
        n0 = _sget(nbv, wid)
        n1 = _sget(nbv, wid + 1)
        e0 = _sget(ebv, wid)
        e1 = _sget(ebv, wid + 1)
        e0a = (e0 // 8) * 8
        nchunks = (e1 - e0a + ce - 1) // ce

        def write_zero_row(ri):
            for c in range(nch):
                otile[ri, pl.ds(c * 16, 16)] = zv
            otile[ri, pl.ds(dpu, 16)] = zv

        def flush_if_full(ri, otb):
            @pl.when(ri == NT - 1)
            def _():
                pltpu.sync_copy(
                    otile, out_hbm.at[pl.ds(pl.multiple_of(otb, 8), NT)])
            return jnp.where(ri == NT - 1, otb + NT, otb)

        def emit_zero(g, otb):
            ri = jnp.clip(g - otb, 0, NT - 1)
            write_zero_row(ri)
            return flush_if_full(ri, otb)

        def finalize(pred, cur, den_v, rw_v, otb):
            ri = jnp.clip(cur - otb, 0, NT - 1)

            @pl.when(pred)
            def _():
                inv_v = 1.0 / (den_v + 1e-16)
                for c in range(nch):
                    otile[ri, pl.ds(c * 16, 16)] = (
                        acc[pl.ds(c * 16, 16)] * inv_v)
                otile[ri, pl.ds(dpu, 16)] = rw_v * inv_v
                @pl.when(ri == NT - 1)
                def _():
                    pltpu.sync_copy(
                        otile, out_hbm.at[pl.ds(pl.multiple_of(otb, 8), NT)])

            return jnp.where(pred & (ri == NT - 1), otb + NT, otb)

        def edge_body(j, carry, ce_base):
            cur, m_s, den_v, rw_v, qwe_s, qtb, otb = carry
            e_glob = ce_base + j
            valid = (e_glob >= e0) & (e_glob < e1)
            d_e = _sget(dbuf, j)
            r_e = _sget(rbuf, j)
            is_new = valid & (d_e != cur)

            otb = finalize(is_new & (cur >= n0), cur, den_v, rw_v, otb)
            gap_lo = jnp.maximum(cur + 1, n0)
            gap_hi = jnp.where(is_new, d_e, gap_lo)
            otb = lax.fori_loop(gap_lo, gap_hi, emit_zero, otb)

            cur2 = jnp.where(is_new, d_e, cur)
            qtb2 = jnp.where(
                is_new & ((cur2 < qtb) | (cur2 >= qtb + QT)),
                (cur2 // QT) * QT, qtb)

            @pl.when(qtb2 != qtb)
            def _():
                pltpu.sync_copy(
                    q_hbm.at[pl.ds(pl.multiple_of(qtb2, 8), QT)], qtile)

            @pl.when(is_new)
            def _():
                for c in range(nch):
                    acc[pl.ds(c * 16, 16)] = zv

            qrow = jnp.clip(cur2 - qtb2, 0, QT - 1)
            pwe = qtile[qrow, pl.ds(0, 16)] * wev[pl.ds(0, 16)]
            for c in range(1, nch):
                pwe = pwe + (qtile[qrow, pl.ds(c * 16, 16)]
                             * wev[pl.ds(c * 16, 16)])
            qwe2 = jnp.where(is_new, jnp.sum(pwe, axis=0), qwe_s)
            m_s = jnp.where(is_new, jnp.float32(-1e30), m_s)
            den_v = jnp.where(is_new, zv, den_v)
            rw_v = jnp.where(is_new, zv, rw_v)

            pk = qtile[qrow, pl.ds(0, 16)] * kbuf[j, pl.ds(0, 16)]
            for c in range(1, nch):
                pk = pk + (qtile[qrow, pl.ds(c * 16, 16)]
                           * kbuf[j, pl.ds(c * 16, 16)])
            a_s = (jnp.sum(pk, axis=0) + r_e * qwe2) * inv_sqrt

            mn_s = jnp.maximum(m_s, a_s)
            e1_v = jnp.exp(jnp.full((16,), m_s - mn_s, jnp.float32))
            w_v = jnp.exp(jnp.full((16,), a_s - mn_s, jnp.float32))

            @pl.when(valid & (a_s > m_s))
            def _():
                for c in range(nch):
                    acc[pl.ds(c * 16, 16)] = acc[pl.ds(c * 16, 16)] * e1_v

            @pl.when(valid)
            def _():
                for c in range(nch):
                    acc[pl.ds(c * 16, 16)] = (
                        acc[pl.ds(c * 16, 16)]
                        + w_v * vbuf[j, pl.ds(c * 16, 16)])

            m2 = jnp.where(valid, mn_s, m_s)
            den2 = jnp.where(valid, den_v * e1_v + w_v, den_v)
            rw2 = jnp.where(valid, rw_v * e1_v + w_v * r_e, rw_v)
            return (cur2, m2, den2, rw2, qwe2, qtb2, otb)

        def chunk_body(cidx, carry):
            ce_base = pl.multiple_of(e0a + cidx * ce, 8)
            pltpu.sync_copy(src_hbm.at[pl.ds(ce_base, ce)], idxv)
            ck = pltpu.async_copy(k_hbm.at[idxv], kbuf, sem1)
            cv = pltpu.async_copy(v_hbm.at[idxv], vbuf, sem2)
            pltpu.sync_copy(dst_hbm.at[pl.ds(ce_base, ce)],
                            dbuf.at[pl.ds(0, ce)])
            pltpu.sync_copy(r_hbm.at[pl.ds(ce_base, ce)],
                            rbuf.at[pl.ds(0, ce)])
            ck.wait()
            cv.wait()
            return lax.fori_loop(
                0, ce, functools.partial(edge_body, ce_base=ce_base), carry)

        carry0 = (n0 - 1, jnp.float32(-1e30), zv, zv, jnp.float32(0.0),
                  jnp.int32(-1024), n0)
        carry = lax.fori_loop(0, nchunks, chunk_body, carry0)
        cur, m_s, den_v, rw_v, qwe_s, qtb, otb = carry

        otb = finalize(cur >= n0, cur, den_v, rw_v, otb)
        otb = lax.fori_loop(jnp.maximum(cur + 1, n0), n1, emit_zero, otb)

    return pl.kernel(
        body,
        out_type=jax.ShapeDtypeStruct((n, wout), jnp.float32),
        mesh=mesh,
        scratch_types=[
            pltpu.VMEM((64,), jnp.int32),
            pltpu.VMEM((64,), jnp.int32),
            pltpu.VMEM((dpu,), jnp.float32),
            pltpu.VMEM((ce,), jnp.int32),
            pltpu.VMEM((ce, dpu_mem), jnp.float32),
            pltpu.VMEM((ce, dpu_mem), jnp.float32),
            pltpu.VMEM((ce + 16,), jnp.int32),
            pltpu.VMEM((ce + 16,), jnp.float32),
            pltpu.VMEM((QT, dpu_mem), jnp.float32),
            pltpu.VMEM((dpu,), jnp.float32),
            pltpu.VMEM((NT, wout), jnp.float32),
            pltpu.SemaphoreType.DMA,
            pltpu.SemaphoreType.DMA,
        ],
        compiler_params=pltpu.CompilerParams(needs_layout_passes=False),
    )


def kernel(t, data, edge_index, params):
    n = data.shape[0]
    e = edge_index.shape[1]
    npad = _rup(n, 1024)
    src = edge_index[0]
    dst = edge_index[1]

    coords = data[:, 0:3]
    dvec = coords[src] - coords[dst]
    r_ij = jnp.sqrt(jnp.sum(dvec * dvec, axis=1) + 1e-12)

    perm = jnp.argsort(dst)
    dst_s = dst[perm]
    src_s = src[perm]
    r_s = r_ij[perm]
    eb_pts = (jnp.arange(1, NW, dtype=jnp.int32) * e) // NW
    nb_mid = (dst_s[eb_pts] // 16) * 16
    nb = jnp.concatenate([
        jnp.zeros((1,), jnp.int32), nb_mid.astype(jnp.int32),
        jnp.full((1,), n, jnp.int32)])
    ebounds = jnp.searchsorted(dst_s, nb).astype(jnp.int32)
    nb48 = jnp.zeros((64,), jnp.int32).at[:NW + 1].set(nb)
    eb48 = jnp.zeros((64,), jnp.int32).at[:NW + 1].set(ebounds)

    e_pad = e + 256
    src_p = jnp.zeros((e_pad,), jnp.int32).at[:e].set(src_s)
    dst_p = jnp.full((e_pad,), n, jnp.int32).at[:e].set(dst_s)
    r_p = jnp.zeros((e_pad,), jnp.float32).at[:e].set(r_s)

    tt = jnp.ones_like(data[:, :1]) * t
    h = jnp.concatenate([tt, data.astype(jnp.float32)], axis=1)

    n_layers = len(params)
    for l, p in enumerate(params):
        Wq, bq, Wk, bk, Wv, bv, We, Ws, bs = p
        din, dout = Wq.shape
        kp = _rup(din, 8)
        dpu = _rup(dout, 16)
        dpu_mem = _rup(dout, 128)
        wout = _rup(dpu + 16, 128)
        ce = 64 if dpu_mem > 256 else 128

        w4 = jnp.zeros((4, kp, dpu_mem), jnp.float32)
        b4 = jnp.zeros((4, 1, dpu_mem), jnp.float32)
        for i, (w, b) in enumerate(
                ((Wq, bq), (Wk, bk), (Wv, bv), (Ws, bs))):
            w4 = w4.at[i, :din, :dout].set(w)
            b4 = b4.at[i, 0, :dout].set(b)
        hp = jnp.zeros((npad, kp), jnp.float32).at[:n, :din].set(h)
        q, kk, v, s = _proj(hp, w4, b4)

        we_pad = jnp.zeros((dpu,), jnp.float32).at[:dout].set(We[0])
        sc = _sc_edge_kernel(n, e_pad, dpu, dpu_mem, wout, ce,
                             float(1.0 / (dout ** 0.5)))
        out_sc = sc(q[:n], kk[:n], v[:n], src_p, dst_p, r_p, we_pad,
                    nb48, eb48)
        attv = out_sc[:, :dout]
        rw = out_sc[:, dpu]
        h = attv + rw[:, None] * We[0][None, :] + s[:n, :dout]
        if l != n_layers - 1:
            h = jax.nn.relu(h)
    return h.astype(jnp.float32)


# speedup vs baseline: 2.4686x; 1.0013x over previous
import functools

import jax
import jax.numpy as jnp
from jax import lax
from jax.experimental import pallas as pl
from jax.experimental.pallas import tpu as pltpu
from jax.experimental.pallas import tpu_sc as plsc

NW = 32
QT = 16
NT = 16


def _rup(x, m):
    return (x + m - 1) // m * m


def _sget(ref, i):
    base = pl.multiple_of((i // 16) * 16, 16)
    v = ref[pl.ds(base, 16)]
    lane = lax.broadcasted_iota(jnp.int32, (16,), 0) == (i - base)
    return jnp.sum(jnp.where(lane, v, jnp.zeros_like(v)), axis=0)


def _proj_body(x_ref, w_ref, b_ref, oq, ok_, ov, os_):
    x = x_ref[...]
    outs = (oq, ok_, ov, os_)
    for i in range(4):
        outs[i][...] = (
            jnp.dot(x, w_ref[i], preferred_element_type=jnp.float32)
            + b_ref[i]
        )


def _proj(x, w4, b4, block_n=1024):
    npad, kp = x.shape
    dp = w4.shape[2]
    grid = npad // block_n
    out_sd = jax.ShapeDtypeStruct((npad, dp), jnp.float32)
    return pl.pallas_call(
        _proj_body,
        grid=(grid,),
        in_specs=[
            pl.BlockSpec((block_n, kp), lambda i: (i, 0)),
            pl.BlockSpec((4, kp, dp), lambda i: (0, 0, 0)),
            pl.BlockSpec((4, 1, dp), lambda i: (0, 0, 0)),
        ],
        out_specs=[pl.BlockSpec((block_n, dp), lambda i: (i, 0))] * 4,
        out_shape=[out_sd] * 4,
    )(x, w4, b4)


def _sc_edge_kernel(n, e_pad, dpu, dpu_mem, wout, ce, inv_sqrt):
    nch = dpu // 16
    mesh = plsc.VectorSubcoreMesh(core_axis_name="c", subcore_axis_name="s")

    def body(q_hbm, k_hbm, v_hbm, src_hbm, dst_hbm, r_hbm, we_hbm,
             nb_hbm, eb_hbm, out_hbm,
             nbv, ebv, wev, idxv, kbuf, vbuf, dbuf, rbuf, qtile, acc,
             otile, sem1, sem2):
        zv = jnp.zeros((16,), jnp.float32)
        wid = lax.axis_index("s") * 2 + lax.axis_index("c")
        pltpu.sync_copy(nb_hbm, nbv)
        pltpu.sync_copy(eb_hbm, ebv)
        pltpu.sync_copy(we_hbm, wev)
        n0 = _sget(nbv, wid)
        n1 = _sget(nbv, wid + 1)
        e0 = _sget(ebv, wid)
        e1 = _sget(ebv, wid + 1)
        e0a = (e0 // 8) * 8
        nchunks = (e1 - e0a + ce - 1) // ce

        def write_zero_row(ri):
            for c in range(nch):
                otile[ri, pl.ds(c * 16, 16)] = zv
            otile[ri, pl.ds(dpu, 16)] = zv

        def flush_if_full(ri, otb):
            @pl.when(ri == NT - 1)
            def _():
                pltpu.sync_copy(
                    otile, out_hbm.at[pl.ds(pl.multiple_of(otb, 8), NT)])
            return jnp.where(ri == NT - 1, otb + NT, otb)

        def emit_zero(g, otb):
            ri = jnp.clip(g - otb, 0, NT - 1)
            write_zero_row(ri)
            return flush_if_full(ri, otb)

        def finalize(pred, cur, den_v, rw_v, otb):
            ri = jnp.clip(cur - otb, 0, NT - 1)

            @pl.when(pred)
            def _():
                inv_v = 1.0 / (den_v + 1e-16)
                for c in range(nch):
                    otile[ri, pl.ds(c * 16, 16)] = (
                        acc[pl.ds(c * 16, 16)] * inv_v)
                otile[ri, pl.ds(dpu, 16)] = rw_v * inv_v
                @pl.when(ri == NT - 1)
                def _():
                    pltpu.sync_copy(
                        otile, out_hbm.at[pl.ds(pl.multiple_of(otb, 8), NT)])

            return jnp.where(pred & (ri == NT - 1), otb + NT, otb)

        def edge_body(j, carry, ce_base):
            cur, m_s, den_v, rw_v, qtb, otb = carry
            e_glob = ce_base + j
            valid = (e_glob >= e0) & (e_glob < e1)
            d_e = _sget(dbuf, j)
            r_e = _sget(rbuf, j)
            r_v = jnp.full((16,), r_e, jnp.float32)
            is_new = valid & (d_e != cur)

            otb = finalize(is_new & (cur >= n0), cur, den_v, rw_v, otb)
            gap_lo = jnp.maximum(cur + 1, n0)
            gap_hi = jnp.where(is_new, d_e, gap_lo)
            otb = lax.fori_loop(gap_lo, gap_hi, emit_zero, otb)

            cur2 = jnp.where(is_new, d_e, cur)
            qtb2 = jnp.where(
                is_new & ((cur2 < qtb) | (cur2 >= qtb + QT)),
                (cur2 // QT) * QT, qtb)

            @pl.when(qtb2 != qtb)
            def _():
                pltpu.sync_copy(
                    q_hbm.at[pl.ds(pl.multiple_of(qtb2, 8), QT)], qtile)

            @pl.when(is_new)
            def _():
                for c in range(nch):
                    acc[pl.ds(c * 16, 16)] = zv

            qrow = jnp.clip(cur2 - qtb2, 0, QT - 1)
            m_s = jnp.where(is_new, jnp.float32(-1e30), m_s)
            den_v = jnp.where(is_new, zv, den_v)
            rw_v = jnp.where(is_new, zv, rw_v)

            pk = qtile[qrow, pl.ds(0, 16)] * (
                kbuf[j, pl.ds(0, 16)] + r_v * wev[pl.ds(0, 16)])
            for c in range(1, nch):
                pk = pk + qtile[qrow, pl.ds(c * 16, 16)] * (
                    kbuf[j, pl.ds(c * 16, 16)] + r_v * wev[pl.ds(c * 16, 16)])
            a_s = jnp.sum(pk, axis=0) * inv_sqrt

            mn_s = jnp.maximum(m_s, a_s)
            e1_v = jnp.exp(jnp.full((16,), m_s - mn_s, jnp.float32))
            w_v = jnp.exp(jnp.full((16,), a_s - mn_s, jnp.float32))

            @pl.when(valid & (a_s > m_s))
            def _():
                for c in range(nch):
                    acc[pl.ds(c * 16, 16)] = acc[pl.ds(c * 16, 16)] * e1_v

            @pl.when(valid)
            def _():
                for c in range(nch):
                    acc[pl.ds(c * 16, 16)] = (
                        acc[pl.ds(c * 16, 16)]
                        + w_v * vbuf[j, pl.ds(c * 16, 16)])

            m2 = jnp.where(valid, mn_s, m_s)
            den2 = jnp.where(valid, den_v * e1_v + w_v, den_v)
            rw2 = jnp.where(valid, rw_v * e1_v + w_v * r_e, rw_v)
            return (cur2, m2, den2, rw2, qtb2, otb)

        def chunk_body(cidx, carry):
            ce_base = pl.multiple_of(e0a + cidx * ce, 8)
            pltpu.sync_copy(src_hbm.at[pl.ds(ce_base, ce)], idxv)
            ck = pltpu.async_copy(k_hbm.at[idxv], kbuf, sem1)
            cv = pltpu.async_copy(v_hbm.at[idxv], vbuf, sem2)
            pltpu.sync_copy(dst_hbm.at[pl.ds(ce_base, ce)],
                            dbuf.at[pl.ds(0, ce)])
            pltpu.sync_copy(r_hbm.at[pl.ds(ce_base, ce)],
                            rbuf.at[pl.ds(0, ce)])
            ck.wait()
            cv.wait()
            return lax.fori_loop(
                0, ce, functools.partial(edge_body, ce_base=ce_base), carry)

        carry0 = (n0 - 1, jnp.float32(-1e30), zv, zv,
                  jnp.int32(-1024), n0)
        carry = lax.fori_loop(0, nchunks, chunk_body, carry0)
        cur, m_s, den_v, rw_v, qtb, otb = carry

        otb = finalize(cur >= n0, cur, den_v, rw_v, otb)
        otb = lax.fori_loop(jnp.maximum(cur + 1, n0), n1, emit_zero, otb)

    return pl.kernel(
        body,
        out_type=jax.ShapeDtypeStruct((n, wout), jnp.float32),
        mesh=mesh,
        scratch_types=[
            pltpu.VMEM((64,), jnp.int32),
            pltpu.VMEM((64,), jnp.int32),
            pltpu.VMEM((dpu,), jnp.float32),
            pltpu.VMEM((ce,), jnp.int32),
            pltpu.VMEM((ce, dpu_mem), jnp.float32),
            pltpu.VMEM((ce, dpu_mem), jnp.float32),
            pltpu.VMEM((ce + 16,), jnp.int32),
            pltpu.VMEM((ce + 16,), jnp.float32),
            pltpu.VMEM((QT, dpu_mem), jnp.float32),
            pltpu.VMEM((dpu,), jnp.float32),
            pltpu.VMEM((NT, wout), jnp.float32),
            pltpu.SemaphoreType.DMA,
            pltpu.SemaphoreType.DMA,
        ],
        compiler_params=pltpu.CompilerParams(needs_layout_passes=False),
    )


def kernel(t, data, edge_index, params):
    n = data.shape[0]
    e = edge_index.shape[1]
    npad = _rup(n, 1024)
    src = edge_index[0]
    dst = edge_index[1]

    coords = data[:, 0:3]
    dvec = coords[src] - coords[dst]
    r_ij = jnp.sqrt(jnp.sum(dvec * dvec, axis=1) + 1e-12)

    perm = jnp.argsort(dst)
    dst_s = dst[perm]
    src_s = src[perm]
    r_s = r_ij[perm]
    eb_pts = (jnp.arange(1, NW, dtype=jnp.int32) * e) // NW
    nb_mid = (dst_s[eb_pts] // 16) * 16
    nb = jnp.concatenate([
        jnp.zeros((1,), jnp.int32), nb_mid.astype(jnp.int32),
        jnp.full((1,), n, jnp.int32)])
    ebounds = jnp.searchsorted(dst_s, nb).astype(jnp.int32)
    nb48 = jnp.zeros((64,), jnp.int32).at[:NW + 1].set(nb)
    eb48 = jnp.zeros((64,), jnp.int32).at[:NW + 1].set(ebounds)

    e_pad = e + 256
    src_p = jnp.zeros((e_pad,), jnp.int32).at[:e].set(src_s)
    dst_p = jnp.full((e_pad,), n, jnp.int32).at[:e].set(dst_s)
    r_p = jnp.zeros((e_pad,), jnp.float32).at[:e].set(r_s)

    tt = jnp.ones_like(data[:, :1]) * t
    h = jnp.concatenate([tt, data.astype(jnp.float32)], axis=1)

    n_layers = len(params)
    for l, p in enumerate(params):
        Wq, bq, Wk, bk, Wv, bv, We, Ws, bs = p
        din, dout = Wq.shape
        kp = _rup(din, 8)
        dpu = _rup(dout, 16)
        dpu_mem = _rup(dout, 128)
        wout = _rup(dpu + 16, 128)
        ce = 64 if dpu_mem > 256 else 128

        w4 = jnp.zeros((4, kp, dpu_mem), jnp.float32)
        b4 = jnp.zeros((4, 1, dpu_mem), jnp.float32)
        for i, (w, b) in enumerate(
                ((Wq, bq), (Wk, bk), (Wv, bv), (Ws, bs))):
            w4 = w4.at[i, :din, :dout].set(w)
            b4 = b4.at[i, 0, :dout].set(b)
        hp = jnp.zeros((npad, kp), jnp.float32).at[:n, :din].set(h)
        q, kk, v, s = _proj(hp, w4, b4)

        we_pad = jnp.zeros((dpu,), jnp.float32).at[:dout].set(We[0])
        sc = _sc_edge_kernel(n, e_pad, dpu, dpu_mem, wout, ce,
                             float(1.0 / (dout ** 0.5)))
        out_sc = sc(q[:n], kk[:n], v[:n], src_p, dst_p, r_p, we_pad,
                    nb48, eb48)
        attv = out_sc[:, :dout]
        rw = out_sc[:, dpu]
        h = attv + rw[:, None] * We[0][None, :] + s[:n, :dout]
        if l != n_layers - 1:
            h = jax.nn.relu(h)
    return h.astype(jnp.float32)
